# baseline (speedup 1.0000x reference)
import functools
import jax
import jax.numpy as jnp
from jax import lax
from jax.experimental import pallas as pl
from jax.experimental.pallas import tpu as pltpu
from jax.experimental.pallas import tpu_sc as plsc

N = 10000
D = 128
K = 5000

NW = 32
NM = 9984
CVR = K // 16
KP = 5120

RSPLIT = 2400
BM = 1000
BM2 = 200


def _make_sc_body(slab0, base_slabs, extra_workers):

    def body(idx_hbm, tpos_hbm, tcol_hbm, a_hbm, at_hbm, h_hbm,
             pa_hbm, hg_hbm,
             idxv, tposv, tcolv, idx40, idx41,
             rows0, rows1, tb0, tb1, outv, hgA, hgB,
             semr0, semr1, semt0, semt1, semh0, semh1, sempa, semhg):
        cid = lax.axis_index("c")
        sid = lax.axis_index("s")
        w = sid * 2 + cid
        nslab = base_slabs + jnp.where(w < extra_workers, 1, 0)
        nt2 = 2 * nslab

        pltpu.sync_copy(idx_hbm, idxv)
        pltpu.sync_copy(tpos_hbm, tposv)
        pltpu.sync_copy(tcol_hbm, tcolv)

        pos16 = jnp.arange(16, dtype=jnp.int32)
        tpos = tposv[pl.ds(0, 16)]
        tcol = tcolv[pl.ds(0, 16)]
        tmask = tpos >= 0
        tposc = jnp.maximum(tpos, 0)

        def sl_of(t2):
            return w + NW * (t2 // 2)

        def j2_of(t2):
            return 2 * (slab0 + sl_of(t2)) + (t2 % 2)

        def start_in(t2, rbuf, tbuf, i4, hgs, u, semr, semt, semh):
            j2 = j2_of(t2)
            rv = plsc.load_gather(idxv, [jnp.minimum(4 * j2 + pos16, K - 1)])
            plsc.store_scatter(i4, [jnp.minimum(pos16, 3)], rv,
                               mask=pos16 < 4)
            pltpu.make_async_copy(
                a_hbm.at[i4, pl.ds(0, NM)], rbuf, semr).start()
            pltpu.make_async_copy(at_hbm.at[i4], tbuf, semt).start()
            pltpu.make_async_copy(
                h_hbm.at[i4], hgs.at[pl.ds(4 * u, 4)], semh).start()

        rvecs = [jnp.full((16,), r, jnp.int32) for r in range(4)]

        def gather_rows(rbuf, tbuf, u):
            @plsc.parallel_loop(0, CVR, unroll=4)
            def _(j):
                off = 16 * j
                cols = idxv[pl.ds(off, 16)]
                for r in range(4):
                    vals = plsc.load_gather(rbuf, [rvecs[r], cols])
                    outv[4 * u + r, pl.ds(off, 16)] = vals

            for r in range(4):
                rvec = rvecs[r]
                R = 4 * u + r
                cpos = jnp.minimum(4992 + pos16, K - 1)
                cols = plsc.load_gather(idxv, [cpos])
                vals = plsc.load_gather(rbuf, [rvec, cols])
                plsc.store_scatter(outv,
                                   [jnp.full((16,), R, jnp.int32), cpos],
                                   vals, mask=pos16 < 8)
                tv = plsc.load_gather(tbuf, [rvec, tcol])
                plsc.store_scatter(outv,
                                   [jnp.full((16,), R, jnp.int32), tposc],
                                   tv, mask=tmask)

        def proc_even(t2, rbuf, tbuf, i4, semr, semt):
            pltpu.make_async_copy(
                a_hbm.at[i4, pl.ds(0, NM)], rbuf, semr).wait()
            pltpu.make_async_copy(at_hbm.at[i4], tbuf, semt).wait()

            @pl.when(t2 > 0)
            def _():
                pltpu.make_async_copy(
                    outv, pa_hbm.at[pl.ds(0, 8)], sempa).wait()

            gather_rows(rbuf, tbuf, 0)

        def proc_odd(t2, rbuf, tbuf, i4, hgs, semr, semt):
            sl = sl_of(t2)
            pltpu.make_async_copy(
                a_hbm.at[i4, pl.ds(0, NM)], rbuf, semr).wait()
            pltpu.make_async_copy(at_hbm.at[i4], tbuf, semt).wait()
            gather_rows(rbuf, tbuf, 1)
            pltpu.make_async_copy(
                outv, pa_hbm.at[pl.ds(8 * sl, 8)], sempa).start()
            pltpu.make_async_copy(
                h_hbm.at[i4], hgs.at[pl.ds(0, 4)], semh0).wait()
            pltpu.make_async_copy(
                h_hbm.at[i4], hgs.at[pl.ds(4, 4)], semh1).wait()
            ch = pltpu.make_async_copy(
                hgs, hg_hbm.at[pl.ds(8 * sl, 8)], semhg)
            ch.start()
            ch.wait()

        start_in(0, rows0, tb0, idx40, hgA, 0, semr0, semt0, semh0)

        def loop_body(t2, _):
            nxt = t2 + 1

            @pl.when(t2 % 4 == 0)
            def _():
                @pl.when(nxt < nt2)
                def _():
                    start_in(nxt, rows1, tb1, idx41, hgA, 1,
                             semr1, semt1, semh1)
                proc_even(t2, rows0, tb0, idx40, semr0, semt0)

            @pl.when(t2 % 4 == 1)
            def _():
                @pl.when(nxt < nt2)
                def _():
                    start_in(nxt, rows0, tb0, idx40, hgB, 0,
                             semr0, semt0, semh0)
                proc_odd(t2, rows1, tb1, idx41, hgA, semr1, semt1)

            @pl.when(t2 % 4 == 2)
            def _():
                @pl.when(nxt < nt2)
                def _():
                    start_in(nxt, rows1, tb1, idx41, hgB, 1,
                             semr1, semt1, semh1)
                proc_even(t2, rows0, tb0, idx40, semr0, semt0)

            @pl.when(t2 % 4 == 3)
            def _():
                @pl.when(nxt < nt2)
                def _():
                    start_in(nxt, rows0, tb0, idx40, hgA, 0,
                             semr0, semt0, semh0)
                proc_odd(t2, rows1, tb1, idx41, hgB, semr1, semt1)

            return 0

        lax.fori_loop(0, nt2, loop_body, 0)
        pltpu.make_async_copy(outv, pa_hbm.at[pl.ds(0, 8)], sempa).wait()

    return body


def _make_sc(n_rows, slab0, base_slabs, extra_workers):
    return functools.partial(
        pl.kernel,
        out_type=(jax.ShapeDtypeStruct((n_rows, KP), jnp.float32),
                  jax.ShapeDtypeStruct((n_rows, D), jnp.float32)),
        mesh=plsc.VectorSubcoreMesh(core_axis_name="c",
                                    subcore_axis_name="s"),
        compiler_params=pltpu.CompilerParams(needs_layout_passes=False),
        scratch_types=[
            pltpu.VMEM((K,), jnp.int32),
            pltpu.VMEM((16,), jnp.int32),
            pltpu.VMEM((16,), jnp.int32),
            pltpu.VMEM((4,), jnp.int32),
            pltpu.VMEM((4,), jnp.int32),
            pltpu.VMEM((4, NM), jnp.float32),
            pltpu.VMEM((4, NM), jnp.float32),
            pltpu.VMEM((4, D), jnp.float32),
            pltpu.VMEM((4, D), jnp.float32),
            pltpu.VMEM((8, KP), jnp.float32),
            pltpu.VMEM((8, D), jnp.float32),
            pltpu.VMEM((8, D), jnp.float32),
            pltpu.SemaphoreType.DMA,
            pltpu.SemaphoreType.DMA,
            pltpu.SemaphoreType.DMA,
            pltpu.SemaphoreType.DMA,
            pltpu.SemaphoreType.DMA,
            pltpu.SemaphoreType.DMA,
            pltpu.SemaphoreType.DMA,
            pltpu.SemaphoreType.DMA,
        ],
    )(_make_sc_body(slab0, base_slabs, extra_workers))


_sc_gather_a = _make_sc(RSPLIT, 0, 9, 12)
_sc_gather_b = _make_sc(K - RSPLIT, RSPLIT // 8, 10, 5)


def _copy_kernel(a_ref, pa_ref):
    pa_ref[...] = a_ref[:, :K]


def _m_kernel(hg_ref, v_ref, w_ref, m_ref):
    m_ref[...] = jnp.dot(hg_ref[...] * v_ref[...], w_ref[...],
                         preferred_element_type=jnp.float32)


def _mm_kernel(a_ref, m_ref, o_ref):
    o_ref[...] = jnp.maximum(
        jnp.dot(a_ref[:, :K], m_ref[...], preferred_element_type=jnp.float32),
        0.0)


def _mmcopy_kernel(pv_ref, a_ref, m_ref, o_ref, pa_ref):
    a = a_ref[:, :K]
    pa_ref[...] = a
    o_ref[...] = jnp.maximum(
        jnp.dot(a, m_ref[...], preferred_element_type=jnp.float32), 0.0)


def kernel(H, A, W, proj_W, proj_b):
    weights = (H @ proj_W + proj_b)[:, 0]
    scores = jax.nn.sigmoid(weights)
    values, idx = jax.lax.top_k(scores, K)

    A_tail = jnp.pad(A[:, NM:], ((0, 0), (0, 112)))
    tpos = jnp.nonzero(idx >= NM, size=16, fill_value=-1)[0].astype(jnp.int32)
    tcol = jnp.where(tpos >= 0, idx[jnp.maximum(tpos, 0)] - NM, 0)
    tcol = tcol.astype(jnp.int32)

    pa_a, hg_a = _sc_gather_a(idx, tpos, tcol, A, A_tail, H)
    pa_b, hg_b = _sc_gather_b(idx, tpos, tcol, A, A_tail, H)

    NB_A = RSPLIT // BM2
    NB_B = (K - RSPLIT) // BM2

    pooled_v1 = pl.pallas_call(
        _copy_kernel,
        grid=(NB_A,),
        in_specs=[pl.BlockSpec((BM2, KP), lambda i: (i, 0))],
        out_specs=pl.BlockSpec((BM2, K), lambda i: (i, 0)),
        out_shape=jax.ShapeDtypeStruct((K, K), jnp.float32),
    )(pa_a)

    Hg = jnp.concatenate([hg_a, hg_b], axis=0)
    M = pl.pallas_call(
        _m_kernel,
        grid=(K // BM,),
        in_specs=[
            pl.BlockSpec((BM, D), lambda i: (i, 0)),
            pl.BlockSpec((BM, 1), lambda i: (i, 0)),
            pl.BlockSpec((D, D), lambda i: (0, 0)),
        ],
        out_specs=pl.BlockSpec((BM, D), lambda i: (i, 0)),
        out_shape=jax.ShapeDtypeStruct((K, D), jnp.float32),
    )(Hg, values[:, None], W)

    out_a = pl.pallas_call(
        _mm_kernel,
        grid=(NB_A,),
        in_specs=[
            pl.BlockSpec((BM2, KP), lambda i: (i, 0)),
            pl.BlockSpec((K, D), lambda i: (0, 0)),
        ],
        out_specs=pl.BlockSpec((BM2, D), lambda i: (i, 0)),
        out_shape=jax.ShapeDtypeStruct((RSPLIT, D), jnp.float32),
    )(pa_a, M)

    out_b, pooled_A = pl.pallas_call(
        _mmcopy_kernel,
        grid=(NB_B,),
        in_specs=[
            pl.BlockSpec(memory_space=pl.ANY),
            pl.BlockSpec((BM2, KP), lambda i: (i, 0)),
            pl.BlockSpec((K, D), lambda i: (0, 0)),
        ],
        out_specs=[
            pl.BlockSpec((BM2, D), lambda i: (i, 0)),
            pl.BlockSpec((BM2, K), lambda i: (i + NB_A, 0)),
        ],
        out_shape=[
            jax.ShapeDtypeStruct((K - RSPLIT, D), jnp.float32),
            jax.ShapeDtypeStruct((K, K), jnp.float32),
        ],
        input_output_aliases={0: 1},
    )(pooled_v1, pa_b, M)

    out = jnp.concatenate([out_a, out_b], axis=0)
    return (out, pooled_A, idx)


# speedup vs baseline: 1.1406x; 1.1406x over previous
import functools
import jax
import jax.numpy as jnp
from jax import lax
from jax.experimental import pallas as pl
from jax.experimental.pallas import tpu as pltpu
from jax.experimental.pallas import tpu_sc as plsc

N = 10000
D = 128
K = 5000

NW = 32
NM = 9984
NSLAB = K // 8
CVR = K // 16
KP = 5120

BM = 1000
BM2 = 200


def _sc_gather_body(idx_hbm, tpos_hbm, tcol_hbm, a_hbm, at_hbm, h_hbm,
                    pa_hbm, hg_hbm,
                    idxv, tposv, tcolv, idx40, idx41,
                    rows0, rows1, tb0, tb1, outv, hgA, hgB,
                    semr0, semr1, semt0, semt1, semh0, semh1, sempa, semhg):
    cid = lax.axis_index("c")
    sid = lax.axis_index("s")
    w = sid * 2 + cid
    nslab = 19 + jnp.where(w < 17, 1, 0)
    nt2 = 2 * nslab

    pltpu.sync_copy(idx_hbm, idxv)
    pltpu.sync_copy(tpos_hbm, tposv)
    pltpu.sync_copy(tcol_hbm, tcolv)

    pos16 = jnp.arange(16, dtype=jnp.int32)
    tpos = tposv[pl.ds(0, 16)]
    tcol = tcolv[pl.ds(0, 16)]
    tmask = tpos >= 0
    tposc = jnp.maximum(tpos, 0)

    def j2_of(t2):
        return 2 * w + 64 * (t2 // 2) + (t2 % 2)

    def slab_of(t2):
        return w + NW * (t2 // 2)

    def start_in(t2, rbuf, tbuf, i4, hgs, u, semr, semt, semh):
        j2 = j2_of(t2)
        rv = plsc.load_gather(idxv, [jnp.minimum(4 * j2 + pos16, K - 1)])
        plsc.store_scatter(i4, [jnp.minimum(pos16, 3)], rv, mask=pos16 < 4)
        pltpu.make_async_copy(
            a_hbm.at[i4, pl.ds(0, NM)], rbuf, semr).start()
        pltpu.make_async_copy(at_hbm.at[i4], tbuf, semt).start()
        pltpu.make_async_copy(
            h_hbm.at[i4], hgs.at[pl.ds(4 * u, 4)], semh).start()

    rvecs = [jnp.full((16,), r, jnp.int32) for r in range(4)]

    def gather_rows(rbuf, tbuf, u):
        @plsc.parallel_loop(0, CVR, unroll=8)
        def _(j):
            off = 16 * j
            cols = idxv[pl.ds(off, 16)]
            for r in range(4):
                vals = plsc.load_gather(rbuf, [rvecs[r], cols])
                outv[4 * u + r, pl.ds(off, 16)] = vals

        for r in range(4):
            rvec = rvecs[r]
            R = 4 * u + r
            cpos = jnp.minimum(4992 + pos16, K - 1)
            cols = plsc.load_gather(idxv, [cpos])
            vals = plsc.load_gather(rbuf, [rvec, cols])
            plsc.store_scatter(outv, [jnp.full((16,), R, jnp.int32), cpos],
                               vals, mask=pos16 < 8)
            tv = plsc.load_gather(tbuf, [rvec, tcol])
            plsc.store_scatter(outv, [jnp.full((16,), R, jnp.int32), tposc],
                               tv, mask=tmask)

    def proc_even(t2, rbuf, tbuf, i4, semr, semt):
        pltpu.make_async_copy(
            a_hbm.at[i4, pl.ds(0, NM)], rbuf, semr).wait()
        pltpu.make_async_copy(at_hbm.at[i4], tbuf, semt).wait()

        @pl.when(t2 > 0)
        def _():
            pltpu.make_async_copy(outv, pa_hbm.at[pl.ds(0, 8)], sempa).wait()

        gather_rows(rbuf, tbuf, 0)

    def proc_odd(t2, rbuf, tbuf, i4, hgs, semr, semt):
        s = slab_of(t2)
        pltpu.make_async_copy(
            a_hbm.at[i4, pl.ds(0, NM)], rbuf, semr).wait()
        pltpu.make_async_copy(at_hbm.at[i4], tbuf, semt).wait()
        gather_rows(rbuf, tbuf, 1)
        pltpu.make_async_copy(outv, pa_hbm.at[pl.ds(8 * s, 8)], sempa).start()
        pltpu.make_async_copy(
            h_hbm.at[i4], hgs.at[pl.ds(0, 4)], semh0).wait()
        pltpu.make_async_copy(
            h_hbm.at[i4], hgs.at[pl.ds(4, 4)], semh1).wait()
        ch = pltpu.make_async_copy(hgs, hg_hbm.at[pl.ds(8 * s, 8)], semhg)
        ch.start()
        ch.wait()

    start_in(0, rows0, tb0, idx40, hgA, 0, semr0, semt0, semh0)

    def body(t2, _):
        nxt = t2 + 1

        @pl.when(t2 % 4 == 0)
        def _():
            @pl.when(nxt < nt2)
            def _():
                start_in(nxt, rows1, tb1, idx41, hgA, 1, semr1, semt1, semh1)
            proc_even(t2, rows0, tb0, idx40, semr0, semt0)

        @pl.when(t2 % 4 == 1)
        def _():
            @pl.when(nxt < nt2)
            def _():
                start_in(nxt, rows0, tb0, idx40, hgB, 0, semr0, semt0, semh0)
            proc_odd(t2, rows1, tb1, idx41, hgA, semr1, semt1)

        @pl.when(t2 % 4 == 2)
        def _():
            @pl.when(nxt < nt2)
            def _():
                start_in(nxt, rows1, tb1, idx41, hgB, 1, semr1, semt1, semh1)
            proc_even(t2, rows0, tb0, idx40, semr0, semt0)

        @pl.when(t2 % 4 == 3)
        def _():
            @pl.when(nxt < nt2)
            def _():
                start_in(nxt, rows0, tb0, idx40, hgA, 0, semr0, semt0, semh0)
            proc_odd(t2, rows1, tb1, idx41, hgB, semr1, semt1)

        return 0

    lax.fori_loop(0, nt2, body, 0)
    pltpu.make_async_copy(outv, pa_hbm.at[pl.ds(0, 8)], sempa).wait()


_sc_gather = functools.partial(
    pl.kernel,
    out_type=(jax.ShapeDtypeStruct((K, KP), jnp.float32),
              jax.ShapeDtypeStruct((K, D), jnp.float32)),
    mesh=plsc.VectorSubcoreMesh(core_axis_name="c", subcore_axis_name="s"),
    compiler_params=pltpu.CompilerParams(needs_layout_passes=False),
    scratch_types=[
        pltpu.VMEM((K,), jnp.int32),
        pltpu.VMEM((16,), jnp.int32),
        pltpu.VMEM((16,), jnp.int32),
        pltpu.VMEM((4,), jnp.int32),
        pltpu.VMEM((4,), jnp.int32),
        pltpu.VMEM((4, NM), jnp.float32),
        pltpu.VMEM((4, NM), jnp.float32),
        pltpu.VMEM((4, D), jnp.float32),
        pltpu.VMEM((4, D), jnp.float32),
        pltpu.VMEM((8, KP), jnp.float32),
        pltpu.VMEM((8, D), jnp.float32),
        pltpu.VMEM((8, D), jnp.float32),
        pltpu.SemaphoreType.DMA,
        pltpu.SemaphoreType.DMA,
        pltpu.SemaphoreType.DMA,
        pltpu.SemaphoreType.DMA,
        pltpu.SemaphoreType.DMA,
        pltpu.SemaphoreType.DMA,
        pltpu.SemaphoreType.DMA,
        pltpu.SemaphoreType.DMA,
    ],
)(_sc_gather_body)


def _m_kernel(hg_ref, v_ref, w_ref, m_ref):
    m_ref[...] = jnp.dot(hg_ref[...] * v_ref[...], w_ref[...],
                         preferred_element_type=jnp.float32)


def _am_kernel(a_ref, m_ref, o_ref, pa_ref):
    a = a_ref[:, :K]
    pa_ref[...] = a
    o_ref[...] = jnp.maximum(
        jnp.dot(a, m_ref[...], preferred_element_type=jnp.float32), 0.0)


def kernel(H, A, W, proj_W, proj_b):
    weights = (H @ proj_W + proj_b)[:, 0]
    scores = jax.nn.sigmoid(weights)
    values, idx = jax.lax.top_k(scores, K)

    A_tail = jnp.pad(A[:, NM:], ((0, 0), (0, 112)))
    tpos = jnp.nonzero(idx >= NM, size=16, fill_value=-1)[0].astype(jnp.int32)
    tcol = jnp.where(tpos >= 0, idx[jnp.maximum(tpos, 0)] - NM, 0)
    tcol = tcol.astype(jnp.int32)

    pa_pad, Hg = _sc_gather(idx, tpos, tcol, A, A_tail, H)

    M = pl.pallas_call(
        _m_kernel,
        grid=(K // BM,),
        in_specs=[
            pl.BlockSpec((BM, D), lambda i: (i, 0)),
            pl.BlockSpec((BM, 1), lambda i: (i, 0)),
            pl.BlockSpec((D, D), lambda i: (0, 0)),
        ],
        out_specs=pl.BlockSpec((BM, D), lambda i: (i, 0)),
        out_shape=jax.ShapeDtypeStruct((K, D), jnp.float32),
    )(Hg, values[:, None], W)

    out, pooled_A = pl.pallas_call(
        _am_kernel,
        grid=(K // BM2,),
        in_specs=[
            pl.BlockSpec((BM2, KP), lambda i: (i, 0)),
            pl.BlockSpec((K, D), lambda i: (0, 0)),
        ],
        out_specs=[
            pl.BlockSpec((BM2, D), lambda i: (i, 0)),
            pl.BlockSpec((BM2, K), lambda i: (i, 0)),
        ],
        out_shape=[
            jax.ShapeDtypeStruct((K, D), jnp.float32),
            jax.ShapeDtypeStruct((K, K), jnp.float32),
        ],
    )(pa_pad, M)

    return (out, pooled_A, idx)


# speedup vs baseline: 1.3057x; 1.1447x over previous
import functools
import jax
import jax.numpy as jnp
from jax import lax
from jax.experimental import pallas as pl
from jax.experimental.pallas import tpu as pltpu
from jax.experimental.pallas import tpu_sc as plsc

N = 10000
D = 128
K = 5000

NW = 32
NM = 9984
NSLAB = K // 8
CVR = K // 16
KP = 5120

BM = 1000
BM2 = 200


def _sc_gather_body(idx_hbm, tpos_hbm, tcol_hbm, a_hbm, at_hbm, h_hbm,
                    pa_hbm, hg_hbm,
                    idxv, tposv, tcolv, idx40, idx41,
                    rows0, rows1, tb0, tb1, outv, hgA, hgB,
                    semr0, semr1, semt0, semt1, semh0, semh1, sempa, semhg):
    cid = lax.axis_index("c")
    sid = lax.axis_index("s")
    w = sid * 2 + cid
    nslab = 19 + jnp.where(w < 17, 1, 0)
    nt2 = 2 * nslab

    pltpu.sync_copy(idx_hbm, idxv)
    pltpu.sync_copy(tpos_hbm, tposv)
    pltpu.sync_copy(tcol_hbm, tcolv)

    pos16 = jnp.arange(16, dtype=jnp.int32)
    tpos = tposv[pl.ds(0, 16)]
    tcol = tcolv[pl.ds(0, 16)]
    tmask = tpos >= 0
    tposc = jnp.maximum(tpos, 0)

    def j2_of(t2):
        return 2 * w + 64 * (t2 // 2) + (t2 % 2)

    def slab_of(t2):
        return w + NW * (t2 // 2)

    def start_in(t2, rbuf, tbuf, i4, hgs, u, semr, semt, semh):
        j2 = j2_of(t2)
        rv = plsc.load_gather(idxv, [jnp.minimum(4 * j2 + pos16, K - 1)])
        plsc.store_scatter(i4, [jnp.minimum(pos16, 3)], rv, mask=pos16 < 4)
        pltpu.make_async_copy(
            a_hbm.at[i4, pl.ds(0, NM)], rbuf, semr).start()
        pltpu.make_async_copy(at_hbm.at[i4], tbuf, semt).start()
        pltpu.make_async_copy(
            h_hbm.at[i4], hgs.at[pl.ds(4 * u, 4)], semh).start()

    rvecs = [jnp.full((16,), r, jnp.int32) for r in range(4)]

    def gather_rows(rbuf, tbuf, u):
        @plsc.parallel_loop(0, CVR, unroll=8)
        def _(j):
            off = 16 * j
            cols = idxv[pl.ds(off, 16)]
            for r in range(4):
                vals = plsc.load_gather(rbuf, [rvecs[r], cols])
                outv[4 * u + r, pl.ds(off, 16)] = vals

        for r in range(4):
            rvec = rvecs[r]
            R = 4 * u + r
            cpos = jnp.minimum(4992 + pos16, K - 1)
            cols = plsc.load_gather(idxv, [cpos])
            vals = plsc.load_gather(rbuf, [rvec, cols])
            plsc.store_scatter(outv, [jnp.full((16,), R, jnp.int32), cpos],
                               vals, mask=pos16 < 8)
            tv = plsc.load_gather(tbuf, [rvec, tcol])
            plsc.store_scatter(outv, [jnp.full((16,), R, jnp.int32), tposc],
                               tv, mask=tmask)

    def proc_even(t2, rbuf, tbuf, i4, semr, semt):
        pltpu.make_async_copy(
            a_hbm.at[i4, pl.ds(0, NM)], rbuf, semr).wait()
        pltpu.make_async_copy(at_hbm.at[i4], tbuf, semt).wait()

        @pl.when(t2 > 0)
        def _():
            pltpu.make_async_copy(outv, pa_hbm.at[pl.ds(0, 8)], sempa).wait()

        gather_rows(rbuf, tbuf, 0)

    def proc_odd(t2, rbuf, tbuf, i4, hgs, semr, semt):
        s = slab_of(t2)
        pltpu.make_async_copy(
            a_hbm.at[i4, pl.ds(0, NM)], rbuf, semr).wait()
        pltpu.make_async_copy(at_hbm.at[i4], tbuf, semt).wait()
        gather_rows(rbuf, tbuf, 1)
        pltpu.make_async_copy(outv, pa_hbm.at[pl.ds(8 * s, 8)], sempa).start()
        pltpu.make_async_copy(
            h_hbm.at[i4], hgs.at[pl.ds(0, 4)], semh0).wait()
        pltpu.make_async_copy(
            h_hbm.at[i4], hgs.at[pl.ds(4, 4)], semh1).wait()
        ch = pltpu.make_async_copy(hgs, hg_hbm.at[pl.ds(8 * s, 8)], semhg)
        ch.start()
        ch.wait()

    start_in(0, rows0, tb0, idx40, hgA, 0, semr0, semt0, semh0)

    def body(t2, _):
        nxt = t2 + 1

        @pl.when(t2 % 4 == 0)
        def _():
            @pl.when(nxt < nt2)
            def _():
                start_in(nxt, rows1, tb1, idx41, hgA, 1, semr1, semt1, semh1)
            proc_even(t2, rows0, tb0, idx40, semr0, semt0)

        @pl.when(t2 % 4 == 1)
        def _():
            @pl.when(nxt < nt2)
            def _():
                start_in(nxt, rows0, tb0, idx40, hgB, 0, semr0, semt0, semh0)
            proc_odd(t2, rows1, tb1, idx41, hgA, semr1, semt1)

        @pl.when(t2 % 4 == 2)
        def _():
            @pl.when(nxt < nt2)
            def _():
                start_in(nxt, rows1, tb1, idx41, hgB, 1, semr1, semt1, semh1)
            proc_even(t2, rows0, tb0, idx40, semr0, semt0)

        @pl.when(t2 % 4 == 3)
        def _():
            @pl.when(nxt < nt2)
            def _():
                start_in(nxt, rows0, tb0, idx40, hgA, 0, semr0, semt0, semh0)
            proc_odd(t2, rows1, tb1, idx41, hgB, semr1, semt1)

        return 0

    lax.fori_loop(0, nt2, body, 0)
    pltpu.make_async_copy(outv, pa_hbm.at[pl.ds(0, 8)], sempa).wait()


_sc_gather = functools.partial(
    pl.kernel,
    out_type=(jax.ShapeDtypeStruct((K, KP), jnp.float32),
              jax.ShapeDtypeStruct((K, D), jnp.float32)),
    mesh=plsc.VectorSubcoreMesh(core_axis_name="c", subcore_axis_name="s"),
    compiler_params=pltpu.CompilerParams(needs_layout_passes=False),
    scratch_types=[
        pltpu.VMEM((K,), jnp.int32),
        pltpu.VMEM((16,), jnp.int32),
        pltpu.VMEM((16,), jnp.int32),
        pltpu.VMEM((4,), jnp.int32),
        pltpu.VMEM((4,), jnp.int32),
        pltpu.VMEM((4, NM), jnp.float32),
        pltpu.VMEM((4, NM), jnp.float32),
        pltpu.VMEM((4, D), jnp.float32),
        pltpu.VMEM((4, D), jnp.float32),
        pltpu.VMEM((8, KP), jnp.float32),
        pltpu.VMEM((8, D), jnp.float32),
        pltpu.VMEM((8, D), jnp.float32),
        pltpu.SemaphoreType.DMA,
        pltpu.SemaphoreType.DMA,
        pltpu.SemaphoreType.DMA,
        pltpu.SemaphoreType.DMA,
        pltpu.SemaphoreType.DMA,
        pltpu.SemaphoreType.DMA,
        pltpu.SemaphoreType.DMA,
        pltpu.SemaphoreType.DMA,
    ],
)(_sc_gather_body)


def _tail_kernel(a_ref, t_ref):
    t_ref[...] = a_ref[...]


def _m_kernel(hg_ref, v_ref, w_ref, m_ref):
    m_ref[...] = jnp.dot(hg_ref[...] * v_ref[...], w_ref[...],
                         preferred_element_type=jnp.float32)


def _am_kernel(a_ref, m_ref, o_ref, pa_ref):
    a = a_ref[:, :K]
    pa_ref[...] = a
    o_ref[...] = jnp.maximum(
        jnp.dot(a, m_ref[...], preferred_element_type=jnp.float32), 0.0)


def kernel(H, A, W, proj_W, proj_b):
    weights = (H @ proj_W + proj_b)[:, 0]
    scores = jax.nn.sigmoid(weights)
    values, idx = jax.lax.top_k(scores, K)

    A_tail = pl.pallas_call(
        _tail_kernel,
        grid=(10,),
        in_specs=[pl.BlockSpec((1000, 128), lambda i: (i, 78))],
        out_specs=pl.BlockSpec((1000, 128), lambda i: (i, 0)),
        out_shape=jax.ShapeDtypeStruct((N, D), jnp.float32),
    )(A)
    eqc = idx[None, :] == (NM + jnp.arange(16, dtype=jnp.int32))[:, None]
    wpos = jnp.arange(1, K + 1, dtype=jnp.float32)
    tpos = (eqc.astype(jnp.float32) @ wpos).astype(jnp.int32) - 1
    tcol = jnp.arange(16, dtype=jnp.int32)

    pa_pad, Hg = _sc_gather(idx, tpos, tcol, A, A_tail, H)

    M = pl.pallas_call(
        _m_kernel,
        grid=(K // BM,),
        in_specs=[
            pl.BlockSpec((BM, D), lambda i: (i, 0)),
            pl.BlockSpec((BM, 1), lambda i: (i, 0)),
            pl.BlockSpec((D, D), lambda i: (0, 0)),
        ],
        out_specs=pl.BlockSpec((BM, D), lambda i: (i, 0)),
        out_shape=jax.ShapeDtypeStruct((K, D), jnp.float32),
    )(Hg, values[:, None], W)

    out, pooled_A = pl.pallas_call(
        _am_kernel,
        grid=(K // BM2,),
        in_specs=[
            pl.BlockSpec((BM2, KP), lambda i: (i, 0)),
            pl.BlockSpec((K, D), lambda i: (0, 0)),
        ],
        out_specs=[
            pl.BlockSpec((BM2, D), lambda i: (i, 0)),
            pl.BlockSpec((BM2, K), lambda i: (i, 0)),
        ],
        out_shape=[
            jax.ShapeDtypeStruct((K, D), jnp.float32),
            jax.ShapeDtypeStruct((K, K), jnp.float32),
        ],
    )(pa_pad, M)

    return (out, pooled_A, idx)
